# baseline (speedup 1.0000x reference)
import jax
import jax.numpy as jnp
from jax.experimental import pallas as pl
from jax.experimental.pallas import tpu as pltpu


def _make_kernel(nb, e_num, s, d, k, out_dim):
    kd = k * d

    def _weight_copies(w1_hbm, w2_hbm, w3_hbm, w1_buf, w2_buf, w3_buf, sems,
                       e_idx, slot):
        return [
            pltpu.make_async_copy(w1_hbm.at[e_idx], w1_buf.at[slot],
                                  sems.at[slot, 0]),
            pltpu.make_async_copy(w2_hbm.at[e_idx], w2_buf.at[slot],
                                  sems.at[slot, 1]),
            pltpu.make_async_copy(w3_hbm.at[e_idx], w3_buf.at[slot],
                                  sems.at[slot, 2]),
        ]

    def _moe_kernel(x_ref, wg_ref, bg_ref, w1_hbm, b1_ref, w2_hbm, b2_ref,
                    w3_hbm, b3_ref, out_ref,
                    inp_ref, w1_buf, w2_buf, w3_buf, sems):
        i = pl.program_id(0)

        @pl.when(i == 0)
        def _start_first():
            for cp in (_weight_copies(w1_hbm, w2_hbm, w3_hbm, w1_buf, w2_buf,
                                      w3_buf, sems, 0, 0)
                       + _weight_copies(w1_hbm, w2_hbm, w3_hbm, w1_buf,
                                        w2_buf, w3_buf, sems,
                                        min(1, e_num - 1), 1)):
                cp.start()

        @pl.when(i < nb)
        def _gate():
            xb = x_ref[0]
            logits = jnp.dot(xb, wg_ref[...],
                             preferred_element_type=jnp.float32)
            logits = logits + bg_ref[...]
            m = jnp.max(logits, axis=0, keepdims=True)
            el = jnp.exp(logits - m)
            denom = jnp.sum(el, axis=0, keepdims=True)
            iota = jax.lax.broadcasted_iota(jnp.int32, el.shape, 0)
            cur = el
            sel_cols = []
            for j in range(k):
                vj = 1.0 if j == 0 else jnp.max(cur, axis=0, keepdims=True)
                aj = jnp.min(jnp.where(cur == vj, iota, s), axis=0,
                             keepdims=True)
                selj = iota == aj
                sel_cols.append(
                    jnp.where(selj, vj / denom, 0.0)[:, :e_num])
                if j < k - 1:
                    cur = jnp.where(selj, -1.0, cur)
            w = jnp.concatenate(sel_cols, axis=1)
            rows = jax.lax.dot_general(
                w, xb, (((0,), (0,)), ((), ())),
                preferred_element_type=jnp.float32)
            for j in range(k):
                inp_ref[pl.ds(i, 1), :, 0, j * d:(j + 1) * d] = (
                    rows[j * e_num:(j + 1) * e_num].reshape(1, e_num, d))

        @pl.when(i >= nb)
        def _mlp():
            ei = i - nb
            slot = jax.lax.rem(ei, 2)

            @pl.when(jnp.logical_and(ei >= 1, ei + 1 < e_num))
            def _start_next():
                nslot = jax.lax.rem(ei + 1, 2)
                for cp in _weight_copies(w1_hbm, w2_hbm, w3_hbm, w1_buf,
                                         w2_buf, w3_buf, sems,
                                         ei + 1, nslot):
                    cp.start()

            for cp in _weight_copies(w1_hbm, w2_hbm, w3_hbm, w1_buf, w2_buf,
                                     w3_buf, sems, ei, slot):
                cp.wait()

            a = inp_ref[:, pl.ds(ei, 1), 0, :].reshape(nb, kd)
            w1c = w1_buf[pl.ds(slot, 1)].reshape(kd, out_dim)
            w2c = w2_buf[pl.ds(slot, 1)].reshape(out_dim, out_dim)
            w3c = w3_buf[pl.ds(slot, 1)].reshape(out_dim, out_dim)
            h = jnp.dot(a, w1c, preferred_element_type=jnp.float32)
            h = jnp.maximum(h + b1_ref[0], 0.0)
            h = jnp.dot(h, w2c, preferred_element_type=jnp.float32)
            h = jnp.maximum(h + b2_ref[0], 0.0)
            h = jnp.dot(h, w3c, preferred_element_type=jnp.float32)
            out_ref[...] = (h + b3_ref[0]).reshape(out_ref.shape)

    return _moe_kernel


def kernel(x, Wg, bg, W1, b1, W2, b2, W3, b3):
    b, s, d = x.shape
    e = Wg.shape[1]
    kd = W1.shape[1]
    k = kd // d
    out_dim = W1.shape[2]

    ep = 128
    wg_p = jnp.zeros((d, ep), dtype=jnp.float32).at[:, :e].set(Wg)
    bg_p = jnp.zeros((1, ep), dtype=jnp.float32).at[0, :e].set(bg)

    b1r = b1.reshape(e, 1, out_dim)
    b2r = b2.reshape(e, 1, out_dim)
    b3r = b3.reshape(e, 1, out_dim)

    def bmap(i):
        return (jnp.maximum(i - b, 0), 0, 0)

    def wmap4(i):
        return (0, jnp.maximum(i - b, 0), 0, 0)

    out = pl.pallas_call(
        _make_kernel(b, e, s, d, k, out_dim),
        grid=(b + e,),
        in_specs=[
            pl.BlockSpec((1, s, d), lambda i: (jnp.minimum(i, b - 1), 0, 0)),
            pl.BlockSpec((d, ep), lambda i: (0, 0)),
            pl.BlockSpec((1, ep), lambda i: (0, 0)),
            pl.BlockSpec(memory_space=pl.ANY),
            pl.BlockSpec((1, 1, out_dim), bmap),
            pl.BlockSpec(memory_space=pl.ANY),
            pl.BlockSpec((1, 1, out_dim), bmap),
            pl.BlockSpec(memory_space=pl.ANY),
            pl.BlockSpec((1, 1, out_dim), bmap),
        ],
        out_specs=pl.BlockSpec((b, 1, 1, out_dim), wmap4),
        out_shape=jax.ShapeDtypeStruct((b, e, 1, out_dim), jnp.float32),
        scratch_shapes=[
            pltpu.VMEM((b, e, 1, kd), jnp.float32),
            pltpu.VMEM((2, kd, out_dim), jnp.float32),
            pltpu.VMEM((2, out_dim, out_dim), jnp.float32),
            pltpu.VMEM((2, out_dim, out_dim), jnp.float32),
            pltpu.SemaphoreType.DMA((2, 3)),
        ],
        compiler_params=pltpu.CompilerParams(
            dimension_semantics=("arbitrary",)),
    )(x, wg_p, bg_p, W1, b1r, W2, b2r, W3, b3r)

    return out.reshape(b, e, out_dim)


# speedup vs baseline: 1.0056x; 1.0056x over previous
import jax
import jax.numpy as jnp
from jax.experimental import pallas as pl
from jax.experimental.pallas import tpu as pltpu


def _make_kernel(nb, e_num, s, d, k, out_dim):
    kd = k * d

    def _weight_copies(w1_hbm, w2_hbm, w3_hbm, w1_buf, w2_buf, w3_buf, sems,
                       e_idx, slot):
        return [
            pltpu.make_async_copy(w1_hbm.at[e_idx], w1_buf.at[slot],
                                  sems.at[slot, 0]),
            pltpu.make_async_copy(w2_hbm.at[e_idx], w2_buf.at[slot],
                                  sems.at[slot, 1]),
            pltpu.make_async_copy(w3_hbm.at[e_idx], w3_buf.at[slot],
                                  sems.at[slot, 2]),
        ]

    def _moe_kernel(x_ref, wg_ref, bg_ref, w1_hbm, b1_ref, w2_hbm, b2_ref,
                    w3_hbm, b3_ref, out_ref,
                    inp_ref, w1_buf, w2_buf, w3_buf, sems):
        i = pl.program_id(0)

        @pl.when(i == 0)
        def _start_first():
            for cp in (_weight_copies(w1_hbm, w2_hbm, w3_hbm, w1_buf, w2_buf,
                                      w3_buf, sems, 0, 0)
                       + _weight_copies(w1_hbm, w2_hbm, w3_hbm, w1_buf,
                                        w2_buf, w3_buf, sems,
                                        min(1, e_num - 1), 1)):
                cp.start()

        @pl.when(i < nb)
        def _gate():
            xb = x_ref[0]
            logits = jnp.dot(xb, wg_ref[...],
                             preferred_element_type=jnp.float32)
            logits = logits + bg_ref[...]
            lgt = jnp.transpose(logits[:, :e_num])
            m = jnp.max(lgt, axis=1, keepdims=True)
            el = jnp.exp(lgt - m)
            denom = jnp.sum(el, axis=1, keepdims=True)
            iota = jax.lax.broadcasted_iota(jnp.int32, el.shape, 1)
            cur = el
            idx_list = []
            p_list = []
            for j in range(k):
                vj = 1.0 if j == 0 else jnp.max(cur, axis=1, keepdims=True)
                aj = jnp.min(jnp.where(cur == vj, iota, s), axis=1,
                             keepdims=True)
                idx_list.append(aj)
                p_list.append(vj / denom)
                if j < k - 1:
                    cur = jnp.where(iota == aj, -1.0, cur)
            iota_s = jax.lax.broadcasted_iota(jnp.int32, (s, e_num), 0)
            w = jnp.concatenate(
                [jnp.where(iota_s == jnp.transpose(idx_list[j]),
                           jnp.transpose(p_list[j]), 0.0)
                 for j in range(k)], axis=1)
            rows = jax.lax.dot_general(
                w, xb, (((0,), (0,)), ((), ())),
                preferred_element_type=jnp.float32)
            for j in range(k):
                inp_ref[pl.ds(i, 1), :, 0, j * d:(j + 1) * d] = (
                    rows[j * e_num:(j + 1) * e_num].reshape(1, e_num, d))

        @pl.when(i >= nb)
        def _mlp():
            ei = i - nb
            slot = jax.lax.rem(ei, 2)

            @pl.when(jnp.logical_and(ei >= 1, ei + 1 < e_num))
            def _start_next():
                nslot = jax.lax.rem(ei + 1, 2)
                for cp in _weight_copies(w1_hbm, w2_hbm, w3_hbm, w1_buf,
                                         w2_buf, w3_buf, sems,
                                         ei + 1, nslot):
                    cp.start()

            for cp in _weight_copies(w1_hbm, w2_hbm, w3_hbm, w1_buf, w2_buf,
                                     w3_buf, sems, ei, slot):
                cp.wait()

            a = inp_ref[:, pl.ds(ei, 1), 0, :].reshape(nb, kd)
            w1c = w1_buf[pl.ds(slot, 1)].reshape(kd, out_dim)
            w2c = w2_buf[pl.ds(slot, 1)].reshape(out_dim, out_dim)
            w3c = w3_buf[pl.ds(slot, 1)].reshape(out_dim, out_dim)
            h = jnp.dot(a, w1c, preferred_element_type=jnp.float32)
            h = jnp.maximum(h + b1_ref[0], 0.0)
            h = jnp.dot(h, w2c, preferred_element_type=jnp.float32)
            h = jnp.maximum(h + b2_ref[0], 0.0)
            h = jnp.dot(h, w3c, preferred_element_type=jnp.float32)
            out_ref[...] = (h + b3_ref[0]).reshape(out_ref.shape)

    return _moe_kernel


def kernel(x, Wg, bg, W1, b1, W2, b2, W3, b3):
    b, s, d = x.shape
    e = Wg.shape[1]
    kd = W1.shape[1]
    k = kd // d
    out_dim = W1.shape[2]

    ep = 128
    wg_p = jnp.zeros((d, ep), dtype=jnp.float32).at[:, :e].set(Wg)
    bg_p = jnp.zeros((1, ep), dtype=jnp.float32).at[0, :e].set(bg)

    b1r = b1.reshape(e, 1, out_dim)
    b2r = b2.reshape(e, 1, out_dim)
    b3r = b3.reshape(e, 1, out_dim)

    def bmap(i):
        return (jnp.maximum(i - b, 0), 0, 0)

    def wmap4(i):
        return (0, jnp.maximum(i - b, 0), 0, 0)

    out = pl.pallas_call(
        _make_kernel(b, e, s, d, k, out_dim),
        grid=(b + e,),
        in_specs=[
            pl.BlockSpec((1, s, d), lambda i: (jnp.minimum(i, b - 1), 0, 0)),
            pl.BlockSpec((d, ep), lambda i: (0, 0)),
            pl.BlockSpec((1, ep), lambda i: (0, 0)),
            pl.BlockSpec(memory_space=pl.ANY),
            pl.BlockSpec((1, 1, out_dim), bmap),
            pl.BlockSpec(memory_space=pl.ANY),
            pl.BlockSpec((1, 1, out_dim), bmap),
            pl.BlockSpec(memory_space=pl.ANY),
            pl.BlockSpec((1, 1, out_dim), bmap),
        ],
        out_specs=pl.BlockSpec((b, 1, 1, out_dim), wmap4),
        out_shape=jax.ShapeDtypeStruct((b, e, 1, out_dim), jnp.float32),
        scratch_shapes=[
            pltpu.VMEM((b, e, 1, kd), jnp.float32),
            pltpu.VMEM((2, kd, out_dim), jnp.float32),
            pltpu.VMEM((2, out_dim, out_dim), jnp.float32),
            pltpu.VMEM((2, out_dim, out_dim), jnp.float32),
            pltpu.SemaphoreType.DMA((2, 3)),
        ],
        compiler_params=pltpu.CompilerParams(
            dimension_semantics=("arbitrary",)),
    )(x, wg_p, bg_p, W1, b1r, W2, b2r, W3, b3r)

    return out.reshape(b, e, out_dim)


# speedup vs baseline: 1.0175x; 1.0119x over previous
import jax
import jax.numpy as jnp
from jax.experimental import pallas as pl
from jax.experimental.pallas import tpu as pltpu


def _make_kernel(nb, e_num, s, d, k, out_dim):
    kd = k * d

    def _weight_copies(w1_hbm, w2_hbm, w3_hbm, w1_buf, w2_buf, w3_buf, sems,
                       e_idx, slot):
        return [
            pltpu.make_async_copy(w1_hbm.at[e_idx], w1_buf.at[slot],
                                  sems.at[slot, 0]),
            pltpu.make_async_copy(w2_hbm.at[e_idx], w2_buf.at[slot],
                                  sems.at[slot, 1]),
            pltpu.make_async_copy(w3_hbm.at[e_idx], w3_buf.at[slot],
                                  sems.at[slot, 2]),
        ]

    def _moe_kernel(x_ref, wg_ref, bg_ref, w1_hbm, b1_ref, w2_hbm, b2_ref,
                    w3_hbm, b3_ref, out_ref,
                    inp_ref, w1_buf, w2_buf, w3_buf, sems):
        i = pl.program_id(0)

        @pl.when(i == 0)
        def _start_first():
            for cp in (_weight_copies(w1_hbm, w2_hbm, w3_hbm, w1_buf, w2_buf,
                                      w3_buf, sems, 0, 0)
                       + _weight_copies(w1_hbm, w2_hbm, w3_hbm, w1_buf,
                                        w2_buf, w3_buf, sems,
                                        min(1, e_num - 1), 1)):
                cp.start()

        @pl.when(i < nb)
        def _gate():
            xb = x_ref[0]
            logits = jnp.dot(xb, wg_ref[...],
                             preferred_element_type=jnp.float32)
            lgt = jnp.transpose(logits[:, :e_num]) + bg_ref[...]
            m = jnp.max(lgt, axis=1, keepdims=True)
            el = jnp.exp(lgt - m)
            denom = jnp.sum(el, axis=1, keepdims=True)
            iota = jax.lax.broadcasted_iota(jnp.int32, el.shape, 1)
            cur = el
            idx_list = []
            p_list = []
            for j in range(k):
                vj = 1.0 if j == 0 else jnp.max(cur, axis=1, keepdims=True)
                aj = jnp.min(jnp.where(cur == vj, iota, s), axis=1,
                             keepdims=True)
                idx_list.append(aj)
                p_list.append(vj / denom)
                if j < k - 1:
                    cur = jnp.where(iota == aj, -1.0, cur)
            iota_s = jax.lax.broadcasted_iota(jnp.int32, (s, e_num), 0)
            w = jnp.concatenate(
                [jnp.where(iota_s == jnp.transpose(idx_list[j]),
                           jnp.transpose(p_list[j]), 0.0)
                 for j in range(k)], axis=1)
            rows = jax.lax.dot_general(
                w, xb, (((0,), (0,)), ((), ())),
                preferred_element_type=jnp.float32)
            for j in range(k):
                inp_ref[pl.ds(i, 1), :, 0, j * d:(j + 1) * d] = (
                    rows[j * e_num:(j + 1) * e_num].reshape(1, e_num, d))

        @pl.when(i >= nb)
        def _mlp():
            ei = i - nb
            slot = jax.lax.rem(ei, 2)

            @pl.when(jnp.logical_and(ei >= 1, ei + 1 < e_num))
            def _start_next():
                nslot = jax.lax.rem(ei + 1, 2)
                for cp in _weight_copies(w1_hbm, w2_hbm, w3_hbm, w1_buf,
                                         w2_buf, w3_buf, sems,
                                         ei + 1, nslot):
                    cp.start()

            for cp in _weight_copies(w1_hbm, w2_hbm, w3_hbm, w1_buf, w2_buf,
                                     w3_buf, sems, ei, slot):
                cp.wait()

            a = inp_ref[:, pl.ds(ei, 1), 0, :].reshape(nb, kd)
            w1c = w1_buf[pl.ds(slot, 1)].reshape(kd, out_dim)
            w2c = w2_buf[pl.ds(slot, 1)].reshape(out_dim, out_dim)
            w3c = w3_buf[pl.ds(slot, 1)].reshape(out_dim, out_dim)
            h = jnp.dot(a, w1c, preferred_element_type=jnp.float32)
            h = jnp.maximum(h + b1_ref[0], 0.0)
            h = jnp.dot(h, w2c, preferred_element_type=jnp.float32)
            h = jnp.maximum(h + b2_ref[0], 0.0)
            h = jnp.dot(h, w3c, preferred_element_type=jnp.float32)
            out_ref[...] = (h + b3_ref[0]).reshape(out_ref.shape)

    return _moe_kernel


def kernel(x, Wg, bg, W1, b1, W2, b2, W3, b3):
    b, s, d = x.shape
    e = Wg.shape[1]
    kd = W1.shape[1]
    k = kd // d
    out_dim = W1.shape[2]

    ep = 128
    wg_p = jnp.zeros((d, ep), dtype=jnp.float32).at[:, :e].set(Wg)
    bg_p = bg.reshape(e, 1)

    b1r = b1.reshape(e, 1, out_dim)
    b2r = b2.reshape(e, 1, out_dim)
    b3r = b3.reshape(e, 1, out_dim)

    def bmap(i):
        return (jnp.maximum(i - b, 0), 0, 0)

    def wmap4(i):
        return (0, jnp.maximum(i - b, 0), 0, 0)

    out = pl.pallas_call(
        _make_kernel(b, e, s, d, k, out_dim),
        grid=(b + e,),
        in_specs=[
            pl.BlockSpec((1, s, d), lambda i: (jnp.minimum(i, b - 1), 0, 0)),
            pl.BlockSpec((d, ep), lambda i: (0, 0)),
            pl.BlockSpec((e, 1), lambda i: (0, 0)),
            pl.BlockSpec(memory_space=pl.ANY),
            pl.BlockSpec((1, 1, out_dim), bmap),
            pl.BlockSpec(memory_space=pl.ANY),
            pl.BlockSpec((1, 1, out_dim), bmap),
            pl.BlockSpec(memory_space=pl.ANY),
            pl.BlockSpec((1, 1, out_dim), bmap),
        ],
        out_specs=pl.BlockSpec((b, 1, 1, out_dim), wmap4),
        out_shape=jax.ShapeDtypeStruct((b, e, 1, out_dim), jnp.float32),
        scratch_shapes=[
            pltpu.VMEM((b, e, 1, kd), jnp.float32),
            pltpu.VMEM((2, kd, out_dim), jnp.float32),
            pltpu.VMEM((2, out_dim, out_dim), jnp.float32),
            pltpu.VMEM((2, out_dim, out_dim), jnp.float32),
            pltpu.SemaphoreType.DMA((2, 3)),
        ],
        compiler_params=pltpu.CompilerParams(
            dimension_semantics=("arbitrary",)),
    )(x, wg_p, bg_p, W1, b1r, W2, b2r, W3, b3r)

    return out.reshape(b, e, out_dim)


# speedup vs baseline: 1.0883x; 1.0696x over previous
import jax
import jax.numpy as jnp
from jax.experimental import pallas as pl
from jax.experimental.pallas import tpu as pltpu


def _make_kernel(nb, e_num, s, d, k, out_dim):
    kd = k * d

    def _weight_copies(w1_hbm, w2_hbm, w3_hbm, w1_buf, w2_buf, w3_buf, sems,
                       e_idx, slot):
        return [
            pltpu.make_async_copy(w1_hbm.at[e_idx], w1_buf.at[slot],
                                  sems.at[slot, 0]),
            pltpu.make_async_copy(w2_hbm.at[e_idx], w2_buf.at[slot],
                                  sems.at[slot, 1]),
            pltpu.make_async_copy(w3_hbm.at[e_idx], w3_buf.at[slot],
                                  sems.at[slot, 2]),
        ]

    def _moe_kernel(x_ref, wg_ref, bg_ref, w1_hbm, b1_ref, w2_hbm, b2_ref,
                    w3_hbm, b3_ref, out_ref,
                    inp_ref, w1_buf, w2_buf, w3_buf, sems):
        i = pl.program_id(0)

        @pl.when(i == 0)
        def _start_first():
            for cp in _weight_copies(w1_hbm, w2_hbm, w3_hbm, w1_buf, w2_buf,
                                     w3_buf, sems, 0, 0):
                cp.start()

        @pl.when(i == nb - 1)
        def _start_second():
            for cp in _weight_copies(w1_hbm, w2_hbm, w3_hbm, w1_buf, w2_buf,
                                     w3_buf, sems, min(1, e_num - 1), 1):
                cp.start()

        @pl.when(i < nb)
        def _gate():
            xb = x_ref[0]
            logits = jnp.dot(xb, wg_ref[...],
                             preferred_element_type=jnp.float32)
            lgt = jnp.transpose(logits[:, :e_num]) + bg_ref[...]
            m = jnp.max(lgt, axis=1, keepdims=True)
            el = jnp.exp(lgt - m)
            denom = jnp.sum(el, axis=1, keepdims=True)
            iota = jax.lax.broadcasted_iota(jnp.int32, el.shape, 1)
            cur = el
            idx_list = []
            p_list = []
            for j in range(k):
                vj = 1.0 if j == 0 else jnp.max(cur, axis=1, keepdims=True)
                aj = jnp.min(jnp.where(cur == vj, iota, s), axis=1,
                             keepdims=True)
                idx_list.append(aj)
                p_list.append(vj / denom)
                if j < k - 1:
                    cur = jnp.where(iota == aj, -1.0, cur)
            iota_s = jax.lax.broadcasted_iota(jnp.int32, (s, e_num), 0)
            w = jnp.concatenate(
                [jnp.where(iota_s == jnp.transpose(idx_list[j]),
                           jnp.transpose(p_list[j]), 0.0)
                 for j in range(k)], axis=1)
            rows = jax.lax.dot_general(
                w, xb, (((0,), (0,)), ((), ())),
                preferred_element_type=jnp.float32)
            for j in range(k):
                inp_ref[pl.ds(i, 1), :, 0, j * d:(j + 1) * d] = (
                    rows[j * e_num:(j + 1) * e_num].reshape(1, e_num, d))

        @pl.when(i >= nb)
        def _mlp():
            ei = i - nb
            slot = jax.lax.rem(ei, 2)

            @pl.when(jnp.logical_and(ei >= 1, ei + 1 < e_num))
            def _start_next():
                nslot = jax.lax.rem(ei + 1, 2)
                for cp in _weight_copies(w1_hbm, w2_hbm, w3_hbm, w1_buf,
                                         w2_buf, w3_buf, sems,
                                         ei + 1, nslot):
                    cp.start()

            for cp in _weight_copies(w1_hbm, w2_hbm, w3_hbm, w1_buf, w2_buf,
                                     w3_buf, sems, ei, slot):
                cp.wait()

            a = inp_ref[:, pl.ds(ei, 1), 0, :].reshape(nb, kd)
            w1c = w1_buf[pl.ds(slot, 1)].reshape(kd, out_dim)
            w2c = w2_buf[pl.ds(slot, 1)].reshape(out_dim, out_dim)
            w3c = w3_buf[pl.ds(slot, 1)].reshape(out_dim, out_dim)
            h = jnp.dot(a, w1c, preferred_element_type=jnp.float32)
            h = jnp.maximum(h + b1_ref[0], 0.0)
            h = jnp.dot(h, w2c, preferred_element_type=jnp.float32)
            h = jnp.maximum(h + b2_ref[0], 0.0)
            h = jnp.dot(h, w3c, preferred_element_type=jnp.float32)
            out_ref[...] = (h + b3_ref[0]).reshape(out_ref.shape)

    return _moe_kernel


def kernel(x, Wg, bg, W1, b1, W2, b2, W3, b3):
    b, s, d = x.shape
    e = Wg.shape[1]
    kd = W1.shape[1]
    k = kd // d
    out_dim = W1.shape[2]

    ep = 128
    wg_p = jnp.zeros((d, ep), dtype=jnp.float32).at[:, :e].set(Wg)
    bg_p = bg.reshape(e, 1)

    b1r = b1.reshape(e, 1, out_dim)
    b2r = b2.reshape(e, 1, out_dim)
    b3r = b3.reshape(e, 1, out_dim)

    def bmap(i):
        return (jnp.maximum(i - b, 0), 0, 0)

    def wmap4(i):
        return (0, jnp.maximum(i - b, 0), 0, 0)

    out = pl.pallas_call(
        _make_kernel(b, e, s, d, k, out_dim),
        grid=(b + e,),
        in_specs=[
            pl.BlockSpec((1, s, d), lambda i: (jnp.minimum(i, b - 1), 0, 0)),
            pl.BlockSpec((d, ep), lambda i: (0, 0)),
            pl.BlockSpec((e, 1), lambda i: (0, 0)),
            pl.BlockSpec(memory_space=pl.ANY),
            pl.BlockSpec((1, 1, out_dim), bmap),
            pl.BlockSpec(memory_space=pl.ANY),
            pl.BlockSpec((1, 1, out_dim), bmap),
            pl.BlockSpec(memory_space=pl.ANY),
            pl.BlockSpec((1, 1, out_dim), bmap),
        ],
        out_specs=pl.BlockSpec((b, 1, 1, out_dim), wmap4),
        out_shape=jax.ShapeDtypeStruct((b, e, 1, out_dim), jnp.float32),
        scratch_shapes=[
            pltpu.VMEM((b, e, 1, kd), jnp.float32),
            pltpu.VMEM((2, kd, out_dim), jnp.float32),
            pltpu.VMEM((2, out_dim, out_dim), jnp.float32),
            pltpu.VMEM((2, out_dim, out_dim), jnp.float32),
            pltpu.SemaphoreType.DMA((2, 3)),
        ],
        compiler_params=pltpu.CompilerParams(
            dimension_semantics=("arbitrary",)),
    )(x, wg_p, bg_p, W1, b1r, W2, b2r, W3, b3r)

    return out.reshape(b, e, out_dim)
